# baseline (speedup 1.0000x reference)
import functools

import jax
import jax.numpy as jnp
from jax.experimental import pallas as pl
from jax.experimental.pallas import tpu as pltpu

_BK = 512


def _segment_step(v_ref, c, c2, argmin_ref, losssum_ref,
                  runval_ref, runchunk_ref, v2_ref,
                  *, m, nchunks, chunk_base, is_first, is_last):
    v = v_ref[...]
    cross2 = jax.lax.dot_general(
        v, c + c, (((1,), (1,)), ((), ())),
        preferred_element_type=jnp.float32)

    @pl.when(is_first)
    def _init():
        v2 = jnp.sum(v * v, axis=1)
        v2_ref[...] = jnp.broadcast_to(v2[:, None], (m, 128))
        runval_ref[...] = jnp.full((m, 128), jnp.inf, jnp.float32)
        runchunk_ref[...] = jnp.zeros((m, 128), jnp.int32)

    v2b = v2_ref[...]
    rv = runval_ref[...]
    rc = runchunk_ref[...]
    for t in range(nchunks):
        sl = slice(t * 128, (t + 1) * 128)
        dist = (v2b + c2[None, sl]) - cross2[:, sl]
        better = dist < rv
        rv = jnp.minimum(rv, dist)
        rc = jnp.where(better, jnp.int32(chunk_base + t), rc)
    runval_ref[...] = rv
    runchunk_ref[...] = rc

    @pl.when(is_last)
    def _finalize():
        gmin = jnp.min(rv, axis=1)
        lane = jax.lax.broadcasted_iota(jnp.int32, (m, 128), 1)
        kidx = rc * 128 + lane
        cand = jnp.where(rv == gmin[:, None], kidx, 2147483647)
        argmin_ref[...] = jnp.min(cand, axis=1)
        losssum_ref[0, 0] = jnp.sum(gmin)


def _vq_kernel(v0_ref, v1_ref, v2_ref, c_ref,
               a0_ref, a1_ref, a2_ref, l0_ref, l1_ref, l2_ref,
               runval_ref, runchunk_ref, vsq_ref):
    j = pl.program_id(0)
    m = v0_ref.shape[0]
    c = c_ref[...]
    c2 = jnp.sum(c * c, axis=1)
    state = (runval_ref, runchunk_ref, vsq_ref)

    @pl.when(j == 0)
    def _seg0():
        _segment_step(v0_ref, c, c2, a0_ref, l0_ref, *state,
                      m=m, nchunks=4, chunk_base=0,
                      is_first=True, is_last=True)

    @pl.when(jnp.logical_and(j >= 1, j <= 2))
    def _seg1():
        _segment_step(v1_ref, c, c2, a1_ref, l1_ref, *state,
                      m=m, nchunks=4, chunk_base=(j - 1) * 4,
                      is_first=j == 1, is_last=j == 2)

    @pl.when(j >= 3)
    def _seg2():
        _segment_step(v2_ref, c, c2, a2_ref, l2_ref, *state,
                      m=m, nchunks=4, chunk_base=(j - 3) * 4,
                      is_first=j == 3, is_last=j == 18)


def kernel(v0, v1, v2, c0, c1, c2):
    b, s, d = v0.shape
    m = b * s
    ccat = jnp.concatenate([c0, c1, c2], axis=0)
    nk = ccat.shape[0] // _BK
    a0, a1, a2, l0, l1, l2 = pl.pallas_call(
        _vq_kernel,
        grid=(nk,),
        in_specs=[
            pl.BlockSpec((m, d), lambda j: (0, 0)),
            pl.BlockSpec((m, d), lambda j: (0, 0)),
            pl.BlockSpec((m, d), lambda j: (0, 0)),
            pl.BlockSpec((_BK, d), lambda j: (j, 0)),
        ],
        out_specs=[
            pl.BlockSpec((m,), lambda j: (0,)),
            pl.BlockSpec((m,), lambda j: (0,)),
            pl.BlockSpec((m,), lambda j: (0,)),
            pl.BlockSpec(memory_space=pltpu.SMEM),
            pl.BlockSpec(memory_space=pltpu.SMEM),
            pl.BlockSpec(memory_space=pltpu.SMEM),
        ],
        out_shape=[
            jax.ShapeDtypeStruct((m,), jnp.int32),
            jax.ShapeDtypeStruct((m,), jnp.int32),
            jax.ShapeDtypeStruct((m,), jnp.int32),
            jax.ShapeDtypeStruct((1, 1), jnp.float32),
            jax.ShapeDtypeStruct((1, 1), jnp.float32),
            jax.ShapeDtypeStruct((1, 1), jnp.float32),
        ],
        scratch_shapes=[
            pltpu.VMEM((m, 128), jnp.float32),
            pltpu.VMEM((m, 128), jnp.int32),
            pltpu.VMEM((m, 128), jnp.float32),
        ],
        compiler_params=pltpu.CompilerParams(
            dimension_semantics=("arbitrary",)),
    )(v0.reshape(m, d), v1.reshape(m, d), v2.reshape(m, d), ccat)
    losses = jnp.stack([l0[0, 0], l1[0, 0], l2[0, 0]]) / jnp.float32(m)
    loss = jnp.mean(losses)
    return (loss, a0.reshape(b, s), a1.reshape(b, s), a2.reshape(b, s))
